# baseline (speedup 1.0000x reference)
import jax
import jax.numpy as jnp
from jax.experimental import pallas as pl
from jax.experimental.pallas import tpu as pltpu


def _copy_body(src_ref, dst_ref):
    dst_ref[...] = src_ref[...]


def _tc_copy(a, block_rows):
    rows, cols = a.shape
    assert rows % block_rows == 0
    return pl.pallas_call(
        _copy_body,
        grid=(rows // block_rows,),
        in_specs=[pl.BlockSpec((block_rows, cols), lambda i: (i, 0))],
        out_specs=pl.BlockSpec((block_rows, cols), lambda i: (i, 0)),
        out_shape=jax.ShapeDtypeStruct(a.shape, a.dtype),
    )(a)


from jax import lax
from jax.experimental.pallas import tpu_sc as plsc


def _sc_copy(a):
    rows, cols = a.shape
    n_workers = 32
    stripe = (rows // (64 * n_workers)) * 64
    tail = rows - stripe * n_workers
    assert tail % 64 == 0
    tail_per = 64
    n_tail_workers = tail // tail_per
    tail_base = stripe * n_workers
    mesh = plsc.VectorSubcoreMesh(core_axis_name="c", subcore_axis_name="s")

    def body(src_hbm, dst_hbm):
        wid = lax.axis_index("s") * 2 + lax.axis_index("c")
        base = wid * stripe
        pltpu.sync_copy(
            src_hbm.at[pl.ds(base, stripe), :],
            dst_hbm.at[pl.ds(base, stripe), :],
        )

        @pl.when(wid < n_tail_workers)
        def _():
            tb = tail_base + wid * tail_per
            pltpu.sync_copy(
                src_hbm.at[pl.ds(tb, tail_per), :],
                dst_hbm.at[pl.ds(tb, tail_per), :],
            )

    return pl.kernel(
        body,
        mesh=mesh,
        out_type=jax.ShapeDtypeStruct(a.shape, a.dtype),
        compiler_params=pltpu.CompilerParams(use_tc_tiling_on_sc=True),
    )(a)


def kernel(x, edge_index, edge_attr):
    del edge_index
    x_out = _tc_copy(x, 1000)
    ea_out = _sc_copy(edge_attr)
    return (x_out, ea_out)


# speedup vs baseline: 18.4478x; 18.4478x over previous
import jax
from jax.experimental import pallas as pl


def _copy_body(src_ref, dst_ref):
    dst_ref[...] = src_ref[...]


def _tc_copy(a, block_rows):
    rows, cols = a.shape
    assert rows % block_rows == 0
    return pl.pallas_call(
        _copy_body,
        grid=(rows // block_rows,),
        in_specs=[pl.BlockSpec((block_rows, cols), lambda i: (i, 0))],
        out_specs=pl.BlockSpec((block_rows, cols), lambda i: (i, 0)),
        out_shape=jax.ShapeDtypeStruct(a.shape, a.dtype),
    )(a)


def kernel(x, edge_index, edge_attr):
    del edge_index
    x_out = _tc_copy(x, 1000)
    ea_out = _tc_copy(edge_attr, 16000)
    return (x_out, ea_out)


# speedup vs baseline: 209.0903x; 11.3342x over previous
import jax
from jax.experimental import pallas as pl


def _copy_body(src_ref, dst_ref):
    dst_ref[...] = src_ref[...]


def _tc_copy(a, block_rows):
    rows, cols = a.shape
    assert rows % block_rows == 0
    return pl.pallas_call(
        _copy_body,
        grid=(rows // block_rows,),
        in_specs=[pl.BlockSpec((block_rows, cols), lambda i: (i, 0))],
        out_specs=pl.BlockSpec((block_rows, cols), lambda i: (i, 0)),
        out_shape=jax.ShapeDtypeStruct(a.shape, a.dtype),
    )(a)


def kernel(x, edge_index, edge_attr):
    del edge_index
    x_out = _tc_copy(x, 1000)
    et = edge_attr.T
    et_out = pl.pallas_call(
        _copy_body,
        grid=(10,),
        in_specs=[pl.BlockSpec((16, 32000), lambda i: (0, i))],
        out_specs=pl.BlockSpec((16, 32000), lambda i: (0, i)),
        out_shape=jax.ShapeDtypeStruct(et.shape, et.dtype),
    )(et)
    return (x_out, et_out.T)


# speedup vs baseline: 270.2774x; 1.2926x over previous
import jax
from jax.experimental import pallas as pl


def _copy_body(src_ref, dst_ref):
    dst_ref[...] = src_ref[...]


def _tc_copy(a, block_rows):
    rows, cols = a.shape
    assert rows % block_rows == 0
    return pl.pallas_call(
        _copy_body,
        grid=(rows // block_rows,),
        in_specs=[pl.BlockSpec((block_rows, cols), lambda i: (i, 0))],
        out_specs=pl.BlockSpec((block_rows, cols), lambda i: (i, 0)),
        out_shape=jax.ShapeDtypeStruct(a.shape, a.dtype),
    )(a)


def _pair_body(x_ref, e_ref, xo_ref, eo_ref):
    xo_ref[...] = x_ref[...]
    eo_ref[...] = e_ref[...]


def kernel(x, edge_index, edge_attr):
    del edge_index
    et = edge_attr.T
    x_out, et_out = pl.pallas_call(
        _pair_body,
        grid=(10,),
        in_specs=[
            pl.BlockSpec((1000, 128), lambda i: (i, 0)),
            pl.BlockSpec((16, 32000), lambda i: (0, i)),
        ],
        out_specs=[
            pl.BlockSpec((1000, 128), lambda i: (i, 0)),
            pl.BlockSpec((16, 32000), lambda i: (0, i)),
        ],
        out_shape=[
            jax.ShapeDtypeStruct(x.shape, x.dtype),
            jax.ShapeDtypeStruct(et.shape, et.dtype),
        ],
    )(x, et)
    return (x_out, et_out.T)


# speedup vs baseline: 291.4723x; 1.0784x over previous
import jax
from jax.experimental import pallas as pl


def _copy_body(src_ref, dst_ref):
    dst_ref[...] = src_ref[...]


def _tc_copy(a, block_rows):
    rows, cols = a.shape
    assert rows % block_rows == 0
    return pl.pallas_call(
        _copy_body,
        grid=(rows // block_rows,),
        in_specs=[pl.BlockSpec((block_rows, cols), lambda i: (i, 0))],
        out_specs=pl.BlockSpec((block_rows, cols), lambda i: (i, 0)),
        out_shape=jax.ShapeDtypeStruct(a.shape, a.dtype),
    )(a)


def _pair_body(x_ref, e_ref, xo_ref, eo_ref):
    xo_ref[...] = x_ref[...]
    eo_ref[...] = e_ref[...]


def kernel(x, edge_index, edge_attr):
    del edge_index
    et = edge_attr.T
    x_out, et_out = pl.pallas_call(
        _pair_body,
        grid=(5,),
        in_specs=[
            pl.BlockSpec((2000, 128), lambda i: (i, 0)),
            pl.BlockSpec((16, 64000), lambda i: (0, i)),
        ],
        out_specs=[
            pl.BlockSpec((2000, 128), lambda i: (i, 0)),
            pl.BlockSpec((16, 64000), lambda i: (0, i)),
        ],
        out_shape=[
            jax.ShapeDtypeStruct(x.shape, x.dtype),
            jax.ShapeDtypeStruct(et.shape, et.dtype),
        ],
    )(x, et)
    return (x_out, et_out.T)


# speedup vs baseline: 318.0458x; 1.0912x over previous
import jax
from jax.experimental import pallas as pl


def _copy_body(src_ref, dst_ref):
    dst_ref[...] = src_ref[...]


def _tc_copy(a, block_rows):
    rows, cols = a.shape
    assert rows % block_rows == 0
    return pl.pallas_call(
        _copy_body,
        grid=(rows // block_rows,),
        in_specs=[pl.BlockSpec((block_rows, cols), lambda i: (i, 0))],
        out_specs=pl.BlockSpec((block_rows, cols), lambda i: (i, 0)),
        out_shape=jax.ShapeDtypeStruct(a.shape, a.dtype),
    )(a)


def _pair_body(x_ref, e_ref, xo_ref, eo_ref):
    xo_ref[...] = x_ref[...]
    eo_ref[...] = e_ref[...]


def kernel(x, edge_index, edge_attr):
    del edge_index
    et = edge_attr.T
    x_out, et_out = pl.pallas_call(
        _pair_body,
        grid=(2,),
        in_specs=[
            pl.BlockSpec((5000, 128), lambda i: (i, 0)),
            pl.BlockSpec((16, 160000), lambda i: (0, i)),
        ],
        out_specs=[
            pl.BlockSpec((5000, 128), lambda i: (i, 0)),
            pl.BlockSpec((16, 160000), lambda i: (0, i)),
        ],
        out_shape=[
            jax.ShapeDtypeStruct(x.shape, x.dtype),
            jax.ShapeDtypeStruct(et.shape, et.dtype),
        ],
    )(x, et)
    return (x_out, et_out.T)
